# Initial kernel scaffold; baseline (speedup 1.0000x reference)
import jax
import jax.numpy as jnp
from jax.experimental import pallas as pl


def kernel(x, edge_index, edge_type, edge_attr, edge_weight, rel_w, enc_w1, enc_b1, enc_w2, enc_b2, bases1, comp1, root1, bias1, bases2, comp2, root2, bias2, bases3, comp3, root3, bias3, gamma1, beta1, gamma2, beta2):
    raise NotImplementedError("write your pallas kernel here")


# speedup vs baseline: 12.6324x; 12.6324x over previous
import jax
import jax.numpy as jnp
from jax import lax
from jax.experimental import pallas as pl
from jax.experimental.pallas import tpu as pltpu
from jax.experimental.pallas import tpu_sc as plsc

N = 10000
E = 320000
R = 8
NR = N * R
D = 128
NBASE = 4
KS = NBASE + 1
NREL = R + 1

NC = 2
NS = 16
NW = NC * NS
EPW = E // NW
C = 80
NCHUNK = EPW // C
ROWS_PT = N // NS
NR_PT = NR // NS

_MESH = dict(mesh=plsc.VectorSubcoreMesh(core_axis_name="c", subcore_axis_name="s"),
             compiler_params=pltpu.CompilerParams(needs_layout_passes=False))

BN = 2000
NBLK = N // BN


def _count_body(key_hbm, zeros_hbm, cnt_out, key_v, ones_v, stage_v, cnt_sp):
    c = lax.axis_index("c")
    s = lax.axis_index("s")
    wid = s * NC + c
    for i in range(C // 16):
        ones_v[pl.ds(16 * i, 16)] = jnp.full((16,), 1.0, jnp.float32)
    pltpu.sync_copy(zeros_hbm, stage_v)
    pltpu.sync_copy(stage_v, cnt_sp.at[pl.ds(s * NR_PT, NR_PT)])
    plsc.subcore_barrier()
    base = wid * EPW

    @pl.loop(0, NCHUNK)
    def _(j):
        eb = base + j * C
        pltpu.sync_copy(key_hbm.at[pl.ds(eb, C)], key_v.at[0])
        pltpu.sync_copy(ones_v, cnt_sp.at[key_v.at[0]], add=True)

    plsc.subcore_barrier()
    pltpu.sync_copy(cnt_sp.at[pl.ds(s * NR_PT, NR_PT)], stage_v)
    pltpu.sync_copy(stage_v, cnt_out.at[pl.ds(c * NR + s * NR_PT, NR_PT)])


def _count_call(key, zeros_nr):
    return pl.kernel(
        _count_body,
        out_type=jax.ShapeDtypeStruct((NC * NR,), jnp.float32),
        scratch_types=[pltpu.VMEM((1, C), jnp.int32),
                       pltpu.VMEM((C,), jnp.float32),
                       pltpu.VMEM((NR_PT,), jnp.float32),
                       pltpu.VMEM_SHARED((NR,), jnp.float32)],
        **_MESH,
    )(key, zeros_nr)


def _scale_body(inv_hbm, key_hbm, scale_out, inv_v, key_v, out_v):
    c = lax.axis_index("c")
    s = lax.axis_index("s")
    wid = s * NC + c
    pltpu.sync_copy(inv_hbm, inv_v)
    base = wid * EPW

    @pl.loop(0, NCHUNK)
    def _(j):
        eb = base + j * C
        pltpu.sync_copy(key_hbm.at[pl.ds(eb, C)], key_v)
        for g in range(C // 16):
            kvec = key_v[pl.ds(16 * g, 16)]
            out_v[pl.ds(16 * g, 16)] = plsc.load_gather(inv_v, [kvec])
        pltpu.sync_copy(out_v, scale_out.at[pl.ds(eb, C)])


def _scale_call(inv, key):
    return pl.kernel(
        _scale_body,
        out_type=jax.ShapeDtypeStruct((E,), jnp.float32),
        scratch_types=[pltpu.VMEM((NR,), jnp.float32),
                       pltpu.VMEM((C,), jnp.int32),
                       pltpu.VMEM((C,), jnp.float32)],
        **_MESH,
    )(inv, key)


N_PAD = 10240
TPT = N_PAD // NS
SROWS = C


def _agg_body(h_hbm, gidx_hbm, dst_hbm, scale_hbm, zeros_hbm, parts_out,
              gidx_v, dst_v, scale_v, rows_v, agg_sp, sem):
    c = lax.axis_index("c")
    s = lax.axis_index("s")
    wid = s * NC + c
    pltpu.sync_copy(zeros_hbm, rows_v)
    for t in range(TPT // SROWS):
        pltpu.sync_copy(rows_v, agg_sp.at[pl.ds(s * TPT + t * SROWS, SROWS)])
    plsc.subcore_barrier()
    base = wid * EPW

    @pl.loop(0, NCHUNK)
    def _(j):
        eb = base + j * C
        pltpu.sync_copy(gidx_hbm.at[pl.ds(eb, C)], gidx_v)
        pltpu.sync_copy(dst_hbm.at[pl.ds(eb, C)], dst_v.at[0])
        pltpu.sync_copy(scale_hbm.at[pl.ds(eb, C)], scale_v)
        pltpu.async_copy(h_hbm.at[gidx_v], rows_v, sem).wait()

        @pl.loop(0, C // 16)
        def _(g):
            svec = scale_v[pl.ds(g * 16, 16)]
            for i in range(16):
                e = g * 16 + i
                si = svec[i]
                for q in range(D // 16):
                    rows_v[e, pl.ds(16 * q, 16)] = rows_v[e, pl.ds(16 * q, 16)] * si

        pltpu.sync_copy(rows_v, agg_sp.at[dst_v.at[0]], add=True)

    plsc.subcore_barrier()
    for t in range(TPT // SROWS):
        r0 = s * TPT + t * SROWS
        pltpu.sync_copy(agg_sp.at[pl.ds(r0, SROWS)], rows_v)
        pltpu.sync_copy(rows_v, parts_out.at[c, pl.ds(r0, SROWS)])


def _agg_call(h_flat, gidx, dst, scale, zeros_nd):
    return pl.kernel(
        _agg_body,
        out_type=jax.ShapeDtypeStruct((NC, N_PAD, D), jnp.float32),
        scratch_types=[pltpu.VMEM((C,), jnp.int32),
                       pltpu.VMEM((1, C), jnp.int32),
                       pltpu.VMEM((C,), jnp.float32),
                       pltpu.VMEM((C, D), jnp.float32),
                       pltpu.VMEM_SHARED((N_PAD, D), jnp.float32),
                       pltpu.SemaphoreType.DMA],
        **_MESH,
    )(h_flat, gidx, dst, scale, zeros_nd)


def _inv_body(cnt_ref, inv_ref):
    tot = cnt_ref[0] + cnt_ref[1]
    inv_ref[...] = 1.0 / jnp.maximum(tot, 1.0)


def _inv_call(cnt_parts):
    out = pl.pallas_call(
        _inv_body,
        out_shape=jax.ShapeDtypeStruct((NR // 128, 128), jnp.float32),
    )(cnt_parts.reshape(NC, NR // 128, 128))
    return out.reshape(NR)


def _combine_w(c_ref, ws_ref):
    acc = c_ref[0, 0, 0] * ws_ref[0]
    for k in range(1, KS):
        acc = acc + c_ref[0, 0, k] * ws_ref[k]
    return acc


def _mm_plain_body(c_ref, ws_ref, x_ref, h_ref):
    w = _combine_w(c_ref, ws_ref)
    h_ref[0] = jnp.dot(x_ref[...], w, preferred_element_type=jnp.float32)


def _mm_plain_call(cstack, wstack, x):
    return pl.pallas_call(
        _mm_plain_body,
        grid=(NBLK, NREL),
        in_specs=[pl.BlockSpec((1, 1, KS), lambda n, r: (r, 0, 0)),
                  pl.BlockSpec((KS, D, D), lambda n, r: (0, 0, 0)),
                  pl.BlockSpec((BN, D), lambda n, r: (n, 0))],
        out_specs=pl.BlockSpec((1, BN, D), lambda n, r: (r, n, 0)),
        out_shape=jax.ShapeDtypeStruct((NREL, N, D), jnp.float32),
    )(cstack, wstack, x)


def _mm_norm_body(c_ref, ws_ref, st_ref, g_ref, b_ref, x_ref, h_ref):
    w = _combine_w(c_ref, ws_ref)
    m = st_ref[0] * (1.0 / N)
    var = st_ref[1] * (1.0 / N) - m * m
    a = lax.rsqrt(var + 1e-5) * g_ref[0]
    xn = (x_ref[...] - m[None, :]) * a[None, :] + b_ref[0][None, :]
    xn = jnp.where(xn >= 0.0, xn, 0.1 * xn)
    h_ref[0] = jnp.dot(xn, w, preferred_element_type=jnp.float32)


def _mm_norm_call(cstack, wstack, stats, gamma, beta, x):
    return pl.pallas_call(
        _mm_norm_body,
        grid=(NBLK, NREL),
        in_specs=[pl.BlockSpec((1, 1, KS), lambda n, r: (r, 0, 0)),
                  pl.BlockSpec((KS, D, D), lambda n, r: (0, 0, 0)),
                  pl.BlockSpec((2, D), lambda n, r: (0, 0)),
                  pl.BlockSpec((1, D), lambda n, r: (0, 0)),
                  pl.BlockSpec((1, D), lambda n, r: (0, 0)),
                  pl.BlockSpec((BN, D), lambda n, r: (n, 0))],
        out_specs=pl.BlockSpec((1, BN, D), lambda n, r: (r, n, 0)),
        out_shape=jax.ShapeDtypeStruct((NREL, N, D), jnp.float32),
    )(cstack, wstack, stats, gamma, beta, x)


def _z_stats_body(p_ref, h_ref, b_ref, z_ref, st_ref):
    z = p_ref[0] + p_ref[1] + h_ref[0] + b_ref[0][None, :]
    z_ref[...] = z
    st = jnp.stack([jnp.sum(z, axis=0), jnp.sum(z * z, axis=0)])

    @pl.when(pl.program_id(0) == 0)
    def _():
        st_ref[...] = st

    @pl.when(pl.program_id(0) != 0)
    def _():
        st_ref[...] = st_ref[...] + st


def _z_stats_call(parts, h_all, bias):
    return pl.pallas_call(
        _z_stats_body,
        grid=(NBLK,),
        in_specs=[pl.BlockSpec((NC, BN, D), lambda n: (0, n, 0)),
                  pl.BlockSpec((1, BN, D), lambda n: (R, n, 0)),
                  pl.BlockSpec((1, D), lambda n: (0, 0))],
        out_specs=[pl.BlockSpec((BN, D), lambda n: (n, 0)),
                   pl.BlockSpec((2, D), lambda n: (0, 0))],
        out_shape=[jax.ShapeDtypeStruct((N, D), jnp.float32),
                   jax.ShapeDtypeStruct((2, D), jnp.float32)],
    )(parts, h_all, bias)


def _final_body(p_ref, h_ref, b_ref, out_ref):
    z = p_ref[0] + p_ref[1] + h_ref[0] + b_ref[0][None, :]
    nrm = jnp.sqrt(jnp.sum(z * z, axis=1, keepdims=True))
    out_ref[...] = z / jnp.maximum(nrm, 1e-12)


def _final_call(parts, h_all, bias):
    return pl.pallas_call(
        _final_body,
        grid=(NBLK,),
        in_specs=[pl.BlockSpec((NC, BN, D), lambda n: (0, n, 0)),
                  pl.BlockSpec((1, BN, D), lambda n: (R, n, 0)),
                  pl.BlockSpec((1, D), lambda n: (0, 0))],
        out_specs=pl.BlockSpec((BN, D), lambda n: (n, 0)),
        out_shape=jax.ShapeDtypeStruct((N, D), jnp.float32),
    )(parts, h_all, bias)


def _stacks(bases, comp, root):
    wstack = jnp.concatenate([bases, root[None]], axis=0)
    cs = jnp.concatenate([comp, jnp.zeros((R, 1), jnp.float32)], axis=1)
    last = jnp.zeros((1, KS), jnp.float32).at[0, NBASE].set(1.0)
    cs = jnp.concatenate([cs, last], axis=0)
    return cs.reshape(NREL, 1, KS), wstack


def kernel(x, edge_index, edge_type, edge_attr, edge_weight,
           rel_w, enc_w1, enc_b1, enc_w2, enc_b2,
           bases1, comp1, root1, bias1,
           bases2, comp2, root2, bias2,
           bases3, comp3, root3, bias3,
           gamma1, beta1, gamma2, beta2):
    src = edge_index[0].astype(jnp.int32)
    dst = edge_index[1].astype(jnp.int32)
    et = edge_type.astype(jnp.int32)
    gidx = et * N + src
    key = dst * R + et
    zeros_nr = jnp.zeros((NR_PT,), jnp.float32)
    zeros_nd = jnp.zeros((C, D), jnp.float32)

    cnt_parts = _count_call(key, zeros_nr)
    inv = _inv_call(cnt_parts)
    scale = _scale_call(inv, key)

    cs1, ws1 = _stacks(bases1, comp1, root1)
    cs2, ws2 = _stacks(bases2, comp2, root2)
    cs3, ws3 = _stacks(bases3, comp3, root3)

    h_all = _mm_plain_call(cs1, ws1, x)
    parts = _agg_call(h_all.reshape(NREL * N, D), gidx, dst, scale, zeros_nd)
    z, stats = _z_stats_call(parts, h_all, bias1.reshape(1, D))

    h_all = _mm_norm_call(cs2, ws2, stats, gamma1.reshape(1, D),
                          beta1.reshape(1, D), z)
    parts = _agg_call(h_all.reshape(NREL * N, D), gidx, dst, scale, zeros_nd)
    z, stats = _z_stats_call(parts, h_all, bias2.reshape(1, D))

    h_all = _mm_norm_call(cs3, ws3, stats, gamma2.reshape(1, D),
                          beta2.reshape(1, D), z)
    parts = _agg_call(h_all.reshape(NREL * N, D), gidx, dst, scale, zeros_nd)
    return _final_call(parts, h_all, bias3.reshape(1, D))
